# baseline (speedup 1.0000x reference)
import functools

import jax
import jax.numpy as jnp
from jax import lax
from jax.experimental import pallas as pl
from jax.experimental.pallas import tpu as pltpu
from jax.experimental.pallas import tpu_sc as plsc

D_MODEL = 64
SCALE = 8.0

NC, NS, L = 2, 16, 16
NW = NC * NS
IDX_MINOR = 128
CH = 512
CHR = CH // IDX_MINOR


def _make_kernel(B, V):
    BPW = B // NW
    IDXR = BPW // IDX_MINOR
    NCH = BPW // CH
    assert BPW % CH == 0 and CH % IDX_MINOR == 0

    mesh = plsc.VectorSubcoreMesh(core_axis_name="c", subcore_axis_name="s")

    @functools.partial(
        pl.kernel,
        out_type=jax.ShapeDtypeStruct((B, D_MODEL), jnp.float32),
        mesh=mesh,
        compiler_params=pltpu.CompilerParams(use_tc_tiling_on_sc=False),
        scratch_types=[
            pltpu.VMEM((IDXR, IDX_MINOR), jnp.int32),
            pltpu.VMEM((CH, D_MODEL), jnp.float32),
            pltpu.VMEM((CH, D_MODEL), jnp.float32),
            pltpu.SemaphoreType.DMA,
            pltpu.SemaphoreType.DMA,
        ],
    )
    def emb(x_hbm, tab_hbm, out_hbm, idx_v, buf0, buf1, gsem0, gsem1):
        wid = lax.axis_index("s") * NC + lax.axis_index("c")
        pltpu.sync_copy(x_hbm.at[pl.ds(wid * IDXR, IDXR)], idx_v)
        out_base = wid * BPW

        def issue(c, buf, sem):
            for j in range(CHR):
                pltpu.async_copy(
                    tab_hbm.at[idx_v.at[c * CHR + j]],
                    buf.at[pl.ds(j * IDX_MINOR, IDX_MINOR)],
                    sem,
                )

        def drain(buf, sem):
            pltpu.make_async_copy(tab_hbm.at[pl.ds(0, CH)], buf, sem).wait()

        bufs = (buf0, buf1)
        sems = (gsem0, gsem1)
        issue(0, buf0, gsem0)

        @pl.loop(0, NCH, step=2)
        def _chunks(g):
            for b in range(2):
                c = g + b
                nb = 1 - b

                @pl.when(c + 1 < NCH)
                def _():
                    issue(c + 1, bufs[nb], sems[nb])

                drain(bufs[b], sems[b])

                @pl.loop(0, CH, unroll=8)
                def _scale(r):
                    for j in range(D_MODEL // L):
                        sl = pl.ds(j * L, L)
                        bufs[b][r, sl] = bufs[b][r, sl] * SCALE

                pltpu.sync_copy(bufs[b], out_hbm.at[pl.ds(out_base + c * CH, CH)])

    return emb


def kernel(x, table):
    BT, T = x.shape
    B = BT * T
    V, D = table.shape
    x2 = x.reshape(B // IDX_MINOR, IDX_MINOR).astype(jnp.int32)
    out = _make_kernel(B, V)(x2, table)
    return out.reshape(BT, T, D)
